# baseline (speedup 1.0000x reference)
import jax
import jax.numpy as jnp
from jax import lax
from jax.experimental import pallas as pl
from jax.experimental.pallas import tpu as pltpu

N_USERS = 1024
N_POSTS = 5000
ROW_BLK = 256
USER_BLK = 256
DAUG = 128
_GRID_POSTS = (N_POSTS + ROW_BLK - 1) // ROW_BLK
_GRID_USERS = N_USERS // USER_BLK


def _leaky_relu(x):
    return jnp.maximum(x, 0.2 * x)


def _elu(x):
    return jnp.where(x > 0, x, jnp.exp(jnp.minimum(x, 0.0)) - 1.0)


def _augment(hw, d):
    r = hw.shape[0]
    return jnp.concatenate(
        [hw, jnp.ones((r, 1), jnp.float32),
         jnp.zeros((r, DAUG - d - 1), jnp.float32)], axis=1)


def _encode_kernel(posts_ref, w_ref, b_ref, out_ref):
    d = w_ref.shape[1]
    hw = jnp.dot(posts_ref[...], w_ref[...],
                 preferred_element_type=jnp.float32,
                 precision=lax.Precision.HIGHEST) + b_ref[...]
    out_ref[...] = _augment(hw, d)


def _gat_rows(adjf, hw_blk, hw_aug, a1_ref, a2_ref, d):
    s1 = jnp.dot(hw_blk, a1_ref[...], preferred_element_type=jnp.float32,
                 precision=lax.Precision.HIGHEST)
    s2 = lax.dot_general(a2_ref[...], hw_aug[:, :d],
                         (((1,), (1,)), ((), ())),
                         preferred_element_type=jnp.float32,
                         precision=lax.Precision.HIGHEST)
    m = _leaky_relu(s1 + jnp.max(s2))
    u1 = s1 - m
    v1 = 0.2 * s1 - m
    s2b = 0.2 * s2
    ex = jnp.exp(jnp.maximum(u1 + s2, v1 + s2b)) * adjf
    acc = jnp.dot(ex, hw_aug, preferred_element_type=jnp.float32)
    l = jnp.maximum(acc[:, d:d + 1], 1e-30)
    return _elu(acc[:, :d] / l)


def _gat1_kernel(adj_ref, hw_blk_ref, hw_aug_ref, a1_ref, a2_ref, wg1_ref,
                 hw1_ref, mask_ref):
    adj = adj_ref[...]
    adjf = adj.astype(jnp.float32)
    d = wg1_ref.shape[0]
    p1 = _gat_rows(adjf, hw_blk_ref[:, :d], hw_aug_ref[...], a1_ref, a2_ref, d)
    hw1 = jnp.dot(p1, wg1_ref[...], preferred_element_type=jnp.float32,
                  precision=lax.Precision.HIGHEST)
    hw1_ref[...] = _augment(hw1, d)
    mask_ref[...] = adj.astype(jnp.int8)


def _gat2_kernel(mask_ref, hw_blk_ref, hw_aug_ref, a1_ref, a2_ref,
                 wp0_ref, bp0_ref, wp1_ref, bp1_ref,
                 p2_ref, label_ref):
    adjf = mask_ref[...].astype(jnp.float32)
    d = wp0_ref.shape[0]
    p2 = _gat_rows(adjf, hw_blk_ref[:, :d], hw_aug_ref[...], a1_ref, a2_ref, d)
    p2_ref[...] = _augment(p2, d)
    t = jnp.maximum(
        jnp.dot(p2, wp0_ref[...], preferred_element_type=jnp.float32,
                precision=lax.Precision.HIGHEST)
        + bp0_ref[...], 0.0)
    label_ref[...] = (jnp.dot(t, wp1_ref[...],
                              preferred_element_type=jnp.float32,
                              precision=lax.Precision.HIGHEST)
                      + bp1_ref[...])


def _user_kernel(users_ref, up_ref, p2_ref, wu_ref, bu_ref,
                 wu0a_ref, wu0b_ref, bu0_ref, wu1_ref, bu1_ref, out_ref):
    d = wu0b_ref.shape[0]
    up = up_ref[...]
    u = jnp.dot(users_ref[...], wu_ref[...],
                preferred_element_type=jnp.float32,
                precision=lax.Precision.HIGHEST) + bu_ref[...]
    acc = jnp.dot(up, p2_ref[...], preferred_element_type=jnp.float32)
    denom = jnp.sum(up, axis=1, keepdims=True) + 1e-9
    agg = acc[:, :d] / denom
    h = jnp.maximum(
        jnp.dot(u, wu0a_ref[...], preferred_element_type=jnp.float32,
                precision=lax.Precision.HIGHEST)
        + jnp.dot(agg, wu0b_ref[...], preferred_element_type=jnp.float32,
                  precision=lax.Precision.HIGHEST)
        + bu0_ref[...], 0.0)
    out_ref[...] = (jnp.dot(h, wu1_ref[...],
                            preferred_element_type=jnp.float32,
                            precision=lax.Precision.HIGHEST)
                    + bu1_ref[...])


def _full(shape):
    return pl.BlockSpec(shape, lambda i: (0,) * len(shape))


def _rows(ncols, blk=ROW_BLK):
    return pl.BlockSpec((blk, ncols), lambda i: (i, 0))


_PARAMS = pltpu.CompilerParams(dimension_semantics=("arbitrary",))


@jax.jit
def kernel(users, posts, post_adjs, up_masking, W_user, b_user, W_post, b_post,
           W_gat0, a1_0, a2_0, W_gat1, a1_1, a2_1,
           Wp0, bp0, Wp1, bp1, Wu0, bu0, Wu1, bu1):
    f32 = jnp.float32
    D = W_gat0.shape[0]

    w_enc = W_post @ W_gat0
    b_enc = (b_post @ W_gat0).reshape(1, D)

    hw0 = pl.pallas_call(
        _encode_kernel,
        grid=(_GRID_POSTS,),
        in_specs=[_rows(posts.shape[1]), _full(w_enc.shape), _full((1, D))],
        out_specs=_rows(DAUG),
        out_shape=jax.ShapeDtypeStruct((N_POSTS, DAUG), f32),
        compiler_params=_PARAMS,
    )(posts, w_enc, b_enc)

    hw1, mask8 = pl.pallas_call(
        _gat1_kernel,
        grid=(_GRID_POSTS,),
        in_specs=[_rows(N_POSTS), _rows(DAUG), _full((N_POSTS, DAUG)),
                  _full((D, 1)), _full((1, D)), _full((D, D))],
        out_specs=[_rows(DAUG), _rows(N_POSTS)],
        out_shape=[jax.ShapeDtypeStruct((N_POSTS, DAUG), f32),
                   jax.ShapeDtypeStruct((N_POSTS, N_POSTS), jnp.int8)],
        compiler_params=_PARAMS,
    )(post_adjs, hw0, hw0, a1_0.reshape(D, 1), a2_0.reshape(1, D), W_gat1)

    p2, post_label = pl.pallas_call(
        _gat2_kernel,
        grid=(_GRID_POSTS,),
        in_specs=[_rows(N_POSTS), _rows(DAUG), _full((N_POSTS, DAUG)),
                  _full((D, 1)), _full((1, D)),
                  _full(Wp0.shape), _full((1, Wp0.shape[1])),
                  _full(Wp1.shape), _full((1, 1))],
        out_specs=[_rows(DAUG), _rows(1)],
        out_shape=[jax.ShapeDtypeStruct((N_POSTS, DAUG), f32),
                   jax.ShapeDtypeStruct((N_POSTS, 1), f32)],
        compiler_params=_PARAMS,
    )(mask8, hw1, hw1, a1_1.reshape(D, 1), a2_1.reshape(1, D),
      Wp0, bp0.reshape(1, -1), Wp1, bp1.reshape(1, 1))

    d_ue = W_user.shape[1]
    user_label = pl.pallas_call(
        _user_kernel,
        grid=(_GRID_USERS,),
        in_specs=[_rows(users.shape[1], USER_BLK), _rows(N_POSTS, USER_BLK),
                  _full((N_POSTS, DAUG)),
                  _full(W_user.shape), _full((1, d_ue)),
                  _full((d_ue, Wu0.shape[1])), _full((D, Wu0.shape[1])),
                  _full((1, Wu0.shape[1])), _full(Wu1.shape), _full((1, 1))],
        out_specs=_rows(1, USER_BLK),
        out_shape=jax.ShapeDtypeStruct((N_USERS, 1), f32),
        compiler_params=_PARAMS,
    )(users, up_masking, p2, W_user, b_user.reshape(1, -1),
      Wu0[:d_ue], Wu0[d_ue:], bu0.reshape(1, -1), Wu1, bu1.reshape(1, 1))

    return (user_label, post_label)


# speedup vs baseline: 1.3632x; 1.3632x over previous
import jax
import jax.numpy as jnp
from jax import lax
from jax.experimental import pallas as pl
from jax.experimental.pallas import tpu as pltpu

N_USERS = 1024
N_POSTS = 5000
ROW_BLK = 256
USER_BLK = 256
DAUG = 128
_GRID_POSTS = (N_POSTS + ROW_BLK - 1) // ROW_BLK
_GRID_USERS = N_USERS // USER_BLK


def _leaky_relu(x):
    return jnp.maximum(x, 0.2 * x)


def _elu(x):
    return jnp.where(x > 0, x, jnp.exp(jnp.minimum(x, 0.0)) - 1.0)


def _augment(hw, d):
    r = hw.shape[0]
    return jnp.concatenate(
        [hw, jnp.ones((r, 1), jnp.float32),
         jnp.zeros((r, DAUG - d - 1), jnp.float32)], axis=1)


def _encode_kernel(posts_ref, w_ref, b_ref, out_ref):
    d = w_ref.shape[1]
    hw = jnp.dot(posts_ref[...], w_ref[...],
                 preferred_element_type=jnp.float32,
                 precision=lax.Precision.HIGHEST) + b_ref[...]
    out_ref[...] = _augment(hw, d)


def _gat_rows(adjf, hw_blk, hw_aug, a1_ref, a2_ref, d):
    s1 = jnp.dot(hw_blk, a1_ref[...], preferred_element_type=jnp.float32,
                 precision=lax.Precision.HIGHEST)
    s2 = lax.dot_general(a2_ref[...], hw_aug[:, :d],
                         (((1,), (1,)), ((), ())),
                         preferred_element_type=jnp.float32)
    m = _leaky_relu(s1 + jnp.max(s2))
    u1 = s1 - m
    v1 = 0.2 * s1 - m
    s2b = 0.2 * s2
    ex = jnp.exp(jnp.maximum(u1 + s2, v1 + s2b)) * adjf
    acc = jnp.dot(ex, hw_aug, preferred_element_type=jnp.float32)
    l = jnp.maximum(acc[:, d:d + 1], 1e-30)
    return _elu(acc[:, :d] / l)


def _gat1_kernel(adj_ref, hw_blk_ref, hw_aug_ref, a1_ref, a2_ref, wg1_ref,
                 hw1_ref, mask_ref):
    adj = adj_ref[...]
    adjf = adj.astype(jnp.float32)
    d = wg1_ref.shape[0]
    p1 = _gat_rows(adjf, hw_blk_ref[:, :d], hw_aug_ref[...], a1_ref, a2_ref, d)
    hw1 = jnp.dot(p1, wg1_ref[...], preferred_element_type=jnp.float32,
                  precision=lax.Precision.HIGHEST)
    hw1_ref[...] = _augment(hw1, d)
    mask_ref[...] = adj.astype(jnp.int8)


def _gat2_kernel(mask_ref, hw_blk_ref, hw_aug_ref, a1_ref, a2_ref,
                 wp0_ref, bp0_ref, wp1_ref, bp1_ref,
                 p2_ref, label_ref):
    adjf = mask_ref[...].astype(jnp.float32)
    d = wp0_ref.shape[0]
    p2 = _gat_rows(adjf, hw_blk_ref[:, :d], hw_aug_ref[...], a1_ref, a2_ref, d)
    p2_ref[...] = _augment(p2, d)
    t = jnp.maximum(
        jnp.dot(p2, wp0_ref[...], preferred_element_type=jnp.float32,
                precision=lax.Precision.HIGHEST)
        + bp0_ref[...], 0.0)
    label_ref[...] = (jnp.dot(t, wp1_ref[...],
                              preferred_element_type=jnp.float32,
                              precision=lax.Precision.HIGHEST)
                      + bp1_ref[...])


def _user_kernel(users_ref, up_ref, p2_ref, wu_ref, bu_ref,
                 wu0a_ref, wu0b_ref, bu0_ref, wu1_ref, bu1_ref, out_ref):
    d = wu0b_ref.shape[0]
    up = up_ref[...]
    u = jnp.dot(users_ref[...], wu_ref[...],
                preferred_element_type=jnp.float32,
                precision=lax.Precision.HIGHEST) + bu_ref[...]
    acc = jnp.dot(up, p2_ref[...], preferred_element_type=jnp.float32)
    denom = jnp.sum(up, axis=1, keepdims=True) + 1e-9
    agg = acc[:, :d] / denom
    h = jnp.maximum(
        jnp.dot(u, wu0a_ref[...], preferred_element_type=jnp.float32,
                precision=lax.Precision.HIGHEST)
        + jnp.dot(agg, wu0b_ref[...], preferred_element_type=jnp.float32,
                  precision=lax.Precision.HIGHEST)
        + bu0_ref[...], 0.0)
    out_ref[...] = (jnp.dot(h, wu1_ref[...],
                            preferred_element_type=jnp.float32,
                            precision=lax.Precision.HIGHEST)
                    + bu1_ref[...])


def _full(shape):
    return pl.BlockSpec(shape, lambda i: (0,) * len(shape))


def _rows(ncols, blk=ROW_BLK):
    return pl.BlockSpec((blk, ncols), lambda i: (i, 0))


_PARAMS = pltpu.CompilerParams(dimension_semantics=("arbitrary",))


@jax.jit
def kernel(users, posts, post_adjs, up_masking, W_user, b_user, W_post, b_post,
           W_gat0, a1_0, a2_0, W_gat1, a1_1, a2_1,
           Wp0, bp0, Wp1, bp1, Wu0, bu0, Wu1, bu1):
    f32 = jnp.float32
    D = W_gat0.shape[0]

    w_enc = W_post @ W_gat0
    b_enc = (b_post @ W_gat0).reshape(1, D)

    hw0 = pl.pallas_call(
        _encode_kernel,
        grid=(_GRID_POSTS,),
        in_specs=[_rows(posts.shape[1]), _full(w_enc.shape), _full((1, D))],
        out_specs=_rows(DAUG),
        out_shape=jax.ShapeDtypeStruct((N_POSTS, DAUG), f32),
        compiler_params=_PARAMS,
    )(posts, w_enc, b_enc)

    hw1, mask8 = pl.pallas_call(
        _gat1_kernel,
        grid=(_GRID_POSTS,),
        in_specs=[_rows(N_POSTS), _rows(DAUG), _full((N_POSTS, DAUG)),
                  _full((D, 1)), _full((1, D)), _full((D, D))],
        out_specs=[_rows(DAUG), _rows(N_POSTS)],
        out_shape=[jax.ShapeDtypeStruct((N_POSTS, DAUG), f32),
                   jax.ShapeDtypeStruct((N_POSTS, N_POSTS), jnp.int8)],
        compiler_params=_PARAMS,
    )(post_adjs, hw0, hw0, a1_0.reshape(D, 1), a2_0.reshape(1, D), W_gat1)

    p2, post_label = pl.pallas_call(
        _gat2_kernel,
        grid=(_GRID_POSTS,),
        in_specs=[_rows(N_POSTS), _rows(DAUG), _full((N_POSTS, DAUG)),
                  _full((D, 1)), _full((1, D)),
                  _full(Wp0.shape), _full((1, Wp0.shape[1])),
                  _full(Wp1.shape), _full((1, 1))],
        out_specs=[_rows(DAUG), _rows(1)],
        out_shape=[jax.ShapeDtypeStruct((N_POSTS, DAUG), f32),
                   jax.ShapeDtypeStruct((N_POSTS, 1), f32)],
        compiler_params=_PARAMS,
    )(mask8, hw1, hw1, a1_1.reshape(D, 1), a2_1.reshape(1, D),
      Wp0, bp0.reshape(1, -1), Wp1, bp1.reshape(1, 1))

    d_ue = W_user.shape[1]
    user_label = pl.pallas_call(
        _user_kernel,
        grid=(_GRID_USERS,),
        in_specs=[_rows(users.shape[1], USER_BLK), _rows(N_POSTS, USER_BLK),
                  _full((N_POSTS, DAUG)),
                  _full(W_user.shape), _full((1, d_ue)),
                  _full((d_ue, Wu0.shape[1])), _full((D, Wu0.shape[1])),
                  _full((1, Wu0.shape[1])), _full(Wu1.shape), _full((1, 1))],
        out_specs=_rows(1, USER_BLK),
        out_shape=jax.ShapeDtypeStruct((N_USERS, 1), f32),
        compiler_params=_PARAMS,
    )(users, up_masking, p2, W_user, b_user.reshape(1, -1),
      Wu0[:d_ue], Wu0[d_ue:], bu0.reshape(1, -1), Wu1, bu1.reshape(1, 1))

    return (user_label, post_label)
